# Initial kernel scaffold; baseline (speedup 1.0000x reference)
import jax
import jax.numpy as jnp
from jax.experimental import pallas as pl


def kernel(input, random_values):
    raise NotImplementedError("write your pallas kernel here")


# speedup vs baseline: 5.3340x; 5.3340x over previous
import functools

import jax
import jax.numpy as jnp
from jax import lax
from jax.experimental import pallas as pl
from jax.experimental.pallas import tpu as pltpu
from jax.experimental.pallas import tpu_sc as plsc

B, C, H, W = 32, 512, 32, 32
S = 512
NC, NS, L = 2, 16, 16
NW = NC * NS
TASKS = B * H
TPW = TASKS // NW


def _body(in_hbm, rv_hbm, out_hbm, in_ref, cs_ref, r_ref, o_ref, sem):
    wid = lax.axis_index("s") * NC + lax.axis_index("c")
    lane = lax.broadcasted_iota(jnp.int32, (L,), 0)

    def task(t, carry):
        b = t // H
        h = t % H
        pltpu.sync_copy(in_hbm.at[b, :, h, :], in_ref)
        pltpu.sync_copy(rv_hbm.at[b, :, h, :], r_ref)

        def csum(c, accs):
            a0 = accs[0] + in_ref[c, pl.ds(0, L)]
            cs_ref[pl.ds(c * W, L)] = a0
            a1 = accs[1] + in_ref[c, pl.ds(L, L)]
            cs_ref[pl.ds(c * W + L, L)] = a1
            return (a0, a1)

        zero = jnp.zeros((L,), jnp.float32)
        totals = lax.fori_loop(0, C, csum, (zero, zero))

        def query(s, _):
            for wg in range(2):
                wvec = lane + wg * L
                t_val = r_ref[s, pl.ds(wg * L, L)] * totals[wg]
                paddr = wvec
                for k in (256, 128, 64, 32, 16, 8, 4, 2, 1):
                    g = plsc.load_gather(cs_ref, [paddr + (k - 1) * W])
                    paddr = paddr + jnp.where(g < t_val, k * W, 0)
                o_ref[s, pl.ds(wg * L, L)] = lax.shift_right_logical(paddr, 5)
            return 0

        lax.fori_loop(0, S, query, 0)
        pltpu.sync_copy(o_ref, out_hbm.at[b, :, h, :])
        return 0

    lax.fori_loop(wid * TPW, (wid + 1) * TPW, task, 0)


@jax.jit
def kernel(input, random_values):
    mesh = plsc.VectorSubcoreMesh(core_axis_name="c", subcore_axis_name="s")
    spikes = pl.kernel(
        _body,
        out_type=jax.ShapeDtypeStruct((B, S, H, W), jnp.int32),
        mesh=mesh,
        compiler_params=pltpu.CompilerParams(
            needs_layout_passes=False, use_tc_tiling_on_sc=False
        ),
        scratch_types=[
            pltpu.VMEM((C, W), jnp.float32),
            pltpu.VMEM((C * W,), jnp.float32),
            pltpu.VMEM((S, W), jnp.float32),
            pltpu.VMEM((S, W), jnp.int32),
            pltpu.SemaphoreType.DMA,
        ],
    )(input, random_values)
    return spikes.astype(jnp.int64)


# speedup vs baseline: 14.7090x; 2.7576x over previous
import functools

import jax
import jax.numpy as jnp
from jax import lax
from jax.experimental import pallas as pl
from jax.experimental.pallas import tpu as pltpu
from jax.experimental.pallas import tpu_sc as plsc

B, C, H, W = 32, 512, 32, 32
S = 512
NC, NS, L = 2, 16, 16
NW = NC * NS
TASKS = B * H
TPW = TASKS // NW


def _body(in_hbm, rv_hbm, out_hbm, in_ref, cs_ref, r_ref, o_ref, sem):
    wid = lax.axis_index("s") * NC + lax.axis_index("c")
    lane = lax.broadcasted_iota(jnp.int32, (L,), 0)

    def task(t, carry):
        b = t // H
        h = t % H
        pltpu.sync_copy(in_hbm.at[b, :, h, :], in_ref)
        pltpu.sync_copy(rv_hbm.at[b, :, h, :], r_ref)

        def csum(i, accs):
            a0, a1 = accs
            for j in range(8):
                c = i * 8 + j
                a0 = a0 + in_ref[c, pl.ds(0, L)]
                cs_ref[pl.ds(c * W, L)] = a0
                a1 = a1 + in_ref[c, pl.ds(L, L)]
                cs_ref[pl.ds(c * W + L, L)] = a1
            return (a0, a1)

        zero = jnp.zeros((L,), jnp.float32)
        totals = lax.fori_loop(0, C // 8, csum, (zero, zero))

        @plsc.parallel_loop(0, S, step=4)
        def query(s0):
            for ds_ in range(4):
                s = s0 + ds_
                for wg in range(2):
                    wvec = lane + wg * L
                    t_val = r_ref[s, pl.ds(wg * L, L)] * totals[wg]
                    paddr = wvec
                    for k in (256, 128, 64, 32, 16, 8, 4, 2, 1):
                        g = plsc.load_gather(cs_ref, [paddr + (k - 1) * W])
                        paddr = paddr + jnp.where(g < t_val, k * W, 0)
                    o_ref[s, pl.ds(wg * L, L)] = lax.shift_right_logical(
                        paddr, 5)
        pltpu.sync_copy(o_ref, out_hbm.at[b, :, h, :])
        return 0

    lax.fori_loop(wid * TPW, (wid + 1) * TPW, task, 0)


@jax.jit
def kernel(input, random_values):
    mesh = plsc.VectorSubcoreMesh(core_axis_name="c", subcore_axis_name="s")
    spikes = pl.kernel(
        _body,
        out_type=jax.ShapeDtypeStruct((B, S, H, W), jnp.int32),
        mesh=mesh,
        compiler_params=pltpu.CompilerParams(
            needs_layout_passes=False, use_tc_tiling_on_sc=False
        ),
        scratch_types=[
            pltpu.VMEM((C, W), jnp.float32),
            pltpu.VMEM((C * W,), jnp.float32),
            pltpu.VMEM((S, W), jnp.float32),
            pltpu.VMEM((S, W), jnp.int32),
            pltpu.SemaphoreType.DMA,
        ],
    )(input, random_values)
    return spikes.astype(jnp.int64)


# speedup vs baseline: 14.8188x; 1.0075x over previous
import functools

import jax
import jax.numpy as jnp
from jax import lax
from jax.experimental import pallas as pl
from jax.experimental.pallas import tpu as pltpu
from jax.experimental.pallas import tpu_sc as plsc

B, C, H, W = 32, 512, 32, 32
S = 512
NC, NS, L = 2, 16, 16
NW = NC * NS
TASKS = B * H
TPW = TASKS // NW


def _body(in_hbm, rv_hbm, out_hbm, in_ref, cs_ref, r_ref, o_ref, sem):
    wid = lax.axis_index("s") * NC + lax.axis_index("c")
    lane = lax.broadcasted_iota(jnp.int32, (L,), 0)

    def task(t, carry):
        b = t // H
        h = t % H
        pltpu.sync_copy(in_hbm.at[b, :, h, :], in_ref)
        pltpu.sync_copy(rv_hbm.at[b, :, h, :], r_ref)

        def csum(i, accs):
            a0, a1 = accs
            for j in range(8):
                c = i * 8 + j
                a0 = a0 + in_ref[c, pl.ds(0, L)]
                cs_ref[pl.ds(c * W, L)] = a0
                a1 = a1 + in_ref[c, pl.ds(L, L)]
                cs_ref[pl.ds(c * W + L, L)] = a1
            return (a0, a1)

        zero = jnp.zeros((L,), jnp.float32)
        totals = lax.fori_loop(0, C // 8, csum, (zero, zero))

        @plsc.parallel_loop(0, S, step=4)
        def query(s0):
            for ds_ in range(4):
                s = s0 + ds_
                for wg in range(2):
                    wvec = lane + wg * L
                    t_val = r_ref[s, pl.ds(wg * L, L)] * totals[wg]
                    q = wvec + 255 * W
                    for k in (256, 128, 64, 32, 16, 8, 4, 2):
                        g = plsc.load_gather(cs_ref, [q])
                        q = q + jnp.where(g < t_val, (k // 2) * W,
                                          -(k // 2) * W)
                    g = plsc.load_gather(cs_ref, [q])
                    pos = lax.shift_right_logical(q, 5) + jnp.where(
                        g < t_val, 1, 0)
                    o_ref[s, pl.ds(wg * L, L)] = pos
        pltpu.sync_copy(o_ref, out_hbm.at[b, :, h, :])
        return 0

    lax.fori_loop(wid * TPW, (wid + 1) * TPW, task, 0)


@jax.jit
def kernel(input, random_values):
    mesh = plsc.VectorSubcoreMesh(core_axis_name="c", subcore_axis_name="s")
    spikes = pl.kernel(
        _body,
        out_type=jax.ShapeDtypeStruct((B, S, H, W), jnp.int32),
        mesh=mesh,
        compiler_params=pltpu.CompilerParams(
            needs_layout_passes=False, use_tc_tiling_on_sc=False
        ),
        scratch_types=[
            pltpu.VMEM((C, W), jnp.float32),
            pltpu.VMEM((C * W,), jnp.float32),
            pltpu.VMEM((S, W), jnp.float32),
            pltpu.VMEM((S, W), jnp.int32),
            pltpu.SemaphoreType.DMA,
        ],
    )(input, random_values)
    return spikes.astype(jnp.int64)


# speedup vs baseline: 28.1346x; 1.8986x over previous
import functools

import jax
import jax.numpy as jnp
from jax import lax
from jax.experimental import pallas as pl
from jax.experimental.pallas import tpu as pltpu
from jax.experimental.pallas import tpu_sc as plsc

B, C, H, W = 32, 512, 32, 32
S = 512
NC, NS, L = 2, 16, 16
NW = NC * NS
TASKS = B * H
TPW = TASKS // NW


def _body(in_hbm, rv_hbm, out_hbm, in_ref, cs_ref, r_ref, o_ref, sem):
    wid = lax.axis_index("s") * NC + lax.axis_index("c")
    lane = lax.broadcasted_iota(jnp.int32, (L,), 0)

    def task(t, carry):
        b = t // 32
        j = (t // 4) % 8
        w0 = (t % 4) * W
        pltpu.sync_copy(in_hbm.at[b, :, j, pl.ds(w0, W)], in_ref)
        pltpu.sync_copy(rv_hbm.at[b, :, j, pl.ds(w0, W)], r_ref)

        def csum(i, accs):
            a0, a1 = accs
            for j in range(8):
                c = i * 8 + j
                a0 = a0 + in_ref[c, pl.ds(0, L)]
                cs_ref[pl.ds(c * W, L)] = a0
                a1 = a1 + in_ref[c, pl.ds(L, L)]
                cs_ref[pl.ds(c * W + L, L)] = a1
            return (a0, a1)

        zero = jnp.zeros((L,), jnp.float32)
        totals = lax.fori_loop(0, C // 8, csum, (zero, zero))

        @plsc.parallel_loop(0, S, step=4)
        def query(s0):
            for ds_ in range(4):
                s = s0 + ds_
                for wg in range(2):
                    wvec = lane + wg * L
                    t_val = r_ref[s, pl.ds(wg * L, L)] * totals[wg]
                    q = wvec + 255 * W
                    for k in (256, 128, 64, 32, 16, 8, 4, 2):
                        g = plsc.load_gather(cs_ref, [q])
                        q = q + jnp.where(g < t_val, (k // 2) * W,
                                          -(k // 2) * W)
                    g = plsc.load_gather(cs_ref, [q])
                    pos = lax.shift_right_logical(q, 5) + jnp.where(
                        g < t_val, 1, 0)
                    o_ref[s, pl.ds(wg * L, L)] = pos
        pltpu.sync_copy(o_ref, out_hbm.at[b, :, j, pl.ds(w0, W)])
        return 0

    lax.fori_loop(wid * TPW, (wid + 1) * TPW, task, 0)


@jax.jit
def kernel(input, random_values):
    mesh = plsc.VectorSubcoreMesh(core_axis_name="c", subcore_axis_name="s")
    x = input.reshape(B, C, 8, 128)
    rv = random_values.reshape(B, S, 8, 128)
    spikes = pl.kernel(
        _body,
        out_type=jax.ShapeDtypeStruct((B, S, 8, 128), jnp.int32),
        mesh=mesh,
        compiler_params=pltpu.CompilerParams(
            needs_layout_passes=False, use_tc_tiling_on_sc=False
        ),
        scratch_types=[
            pltpu.VMEM((C, W), jnp.float32),
            pltpu.VMEM((C * W,), jnp.float32),
            pltpu.VMEM((S, W), jnp.float32),
            pltpu.VMEM((S, W), jnp.int32),
            pltpu.SemaphoreType.DMA,
        ],
    )(x, rv)
    return spikes.reshape(B, S, H, W).astype(jnp.int64)
